# baseline (speedup 1.0000x reference)
import jax
import jax.numpy as jnp
from jax import lax
from jax.experimental import pallas as pl
from jax.experimental.pallas import tpu as pltpu

CH = 128
UPC = 8 * 8 * 9
CUF = 640
VMEM_LIMIT = 60 * 1024 * 1024


def _make_body(C, H, W, TH):
    HW = H * W
    hpW = (H + 2) * W
    THW = TH * W

    def _body(x_ref, w1_ref, b1_ref, w2_ref, b2_ref, wf_ref, bf_ref, be_ref,
              eta_ref, up_ref, h2_ref, acc_ref):
        t = pl.program_id(1)

        def convT(xp, w_ref, b_ref, m0, m2):
            zcol = jnp.zeros((C, 1), jnp.bfloat16)
            xr = jnp.concatenate([zcol, xp[:, :hpW - 1]], axis=1)
            xr = jnp.where(m0, xr, jnp.bfloat16(0))
            xl = jnp.concatenate([xp[:, 1:], zcol], axis=1)
            xl = jnp.where(m2, xl, jnp.bfloat16(0))
            xs = jnp.concatenate([xr, xp, xl], axis=0)
            acc = jnp.zeros((C, HW), jnp.float32)
            for dy in range(3):
                acc = acc + jnp.dot(w_ref[dy],
                                    xs[:, dy * W:dy * W + HW],
                                    preferred_element_type=jnp.float32)
            return acc + b_ref[...]

        @pl.when(t <= 1)
        def _():
            posw = lax.broadcasted_iota(jnp.int32, (1, hpW), 1) % W
            m0 = posw != 0
            m2 = posw != W - 1
            zrow = jnp.zeros((C, W), jnp.bfloat16)
            xf = x_ref[0, 0].astype(jnp.bfloat16).reshape(C, HW)
            xp = jnp.concatenate([zrow, xf, zrow], axis=1)
            h1 = jnp.maximum(convT(xp, w1_ref, b1_ref, m0, m2), 0.0)

            @pl.when(t == 0)
            def _():
                acc_ref[...] = h1

            @pl.when(t == 1)
            def _():
                gm = ((acc_ref[...] + h1) * 0.5).astype(jnp.bfloat16)
                gp = jnp.concatenate([zrow, gm, zrow], axis=1)
                h2 = jnp.maximum(convT(gp, w2_ref, b2_ref, m0, m2), 0.0)
                h2_ref[:, 0:W] = zrow
                h2_ref[:, W:W + HW] = h2.astype(jnp.bfloat16)
                h2_ref[:, W + HW:] = zrow

        @pl.when(t >= 1)
        def _():
            tt = t - 1
            r0W = pl.multiple_of(tt * THW, THW)
            hs = h2_ref[:, pl.ds(r0W, THW + 2 * W)]
            f = jnp.dot(wf_ref[...], hs,
                        preferred_element_type=jnp.float32) + bf_ref[...]
            up_ref[...] = f[:UPC, W:W + THW].reshape(1, 1, UPC, TH, W)

            E = f[UPC:UPC + 9, :]
            zc9 = jnp.zeros((9, 1), jnp.float32)
            Ep = jnp.concatenate([zc9, E, zc9], axis=1)
            lane = lax.broadcasted_iota(jnp.int32, (1, THW), 1) % W
            acc9 = jnp.zeros((1, THW), jnp.float32)
            for k in range(9):
                dy, dx = divmod(k, 3)
                s = Ep[k:k + 1, dy * W + dx:dy * W + dx + THW]
                if dx == 0:
                    s = jnp.where(lane != 0, s, 0.0)
                elif dx == 2:
                    s = jnp.where(lane != W - 1, s, 0.0)
                acc9 = acc9 + s
            z = acc9 + be_ref[0, 0]
            sp = jnp.where(z > 20.0, z,
                           jnp.log1p(jnp.exp(jnp.minimum(z, 20.0))))
            eta_ref[...] = (0.01 * sp).reshape(1, 1, TH, W)

    return _body


def kernel(net, w1, b1, w2, b2, we, be, wu, bu):
    B, num, C, H, W = net.shape
    G = num // 2
    BG = B * G
    TH = 8 if H % 8 == 0 else H
    NT = H // TH

    w1t = w1.reshape(3, 3, C, C).transpose(0, 3, 1, 2) \
            .reshape(3, C, 3 * C).astype(jnp.bfloat16)
    w2t = w2.reshape(3, 3, C, C).transpose(0, 3, 1, 2) \
            .reshape(3, C, 3 * C).astype(jnp.bfloat16)
    b1c = b1.reshape(C, 1)
    b2c = b2.reshape(C, 1)
    wft = jnp.zeros((CUF, C), jnp.float32)
    wft = wft.at[:UPC].set(wu.T).at[UPC:UPC + 9].set(we.reshape(9, C))
    wft = wft.astype(jnp.bfloat16)
    bfc = jnp.zeros((CUF, 1), jnp.float32).at[:UPC, 0].set(bu[0])

    eta, up = pl.pallas_call(
        _make_body(C, H, W, TH),
        out_shape=(
            jax.ShapeDtypeStruct((B, G, H, W), jnp.float32),
            jax.ShapeDtypeStruct((B, G, UPC, H, W), jnp.float32),
        ),
        grid=(BG, NT + 1),
        in_specs=[
            pl.BlockSpec(
                (1, 1, C, H, W),
                lambda g, t: (g // G, 2 * (g % G) + jnp.minimum(t, 1),
                              0, 0, 0)),
            pl.BlockSpec((3, C, 3 * C), lambda g, t: (0, 0, 0)),
            pl.BlockSpec((C, 1), lambda g, t: (0, 0)),
            pl.BlockSpec((3, C, 3 * C), lambda g, t: (0, 0, 0)),
            pl.BlockSpec((C, 1), lambda g, t: (0, 0)),
            pl.BlockSpec((CUF, C), lambda g, t: (0, 0)),
            pl.BlockSpec((CUF, 1), lambda g, t: (0, 0)),
            pl.BlockSpec((1, 1), lambda g, t: (0, 0)),
        ],
        out_specs=[
            pl.BlockSpec(
                (1, 1, TH, W),
                lambda g, t: (g // G, g % G, jnp.maximum(t - 1, 0), 0)),
            pl.BlockSpec(
                (1, 1, UPC, TH, W),
                lambda g, t: (g // G, g % G, 0, jnp.maximum(t - 1, 0), 0)),
        ],
        scratch_shapes=[pltpu.VMEM((C, (H + 2) * W), jnp.bfloat16),
                        pltpu.VMEM((C, H * W), jnp.float32)],
        compiler_params=pltpu.CompilerParams(
            dimension_semantics=("parallel", "arbitrary"),
            vmem_limit_bytes=VMEM_LIMIT),
    )(net, w1t, b1c, w2t, b2c, wft, bfc, be)

    return eta, up


# speedup vs baseline: 1.1333x; 1.1333x over previous
import jax
import jax.numpy as jnp
from jax import lax
from jax.experimental import pallas as pl
from jax.experimental.pallas import tpu as pltpu

CH = 128
UPC = 8 * 8 * 9
CUF = 640
VMEM_LIMIT = 60 * 1024 * 1024


def _make_body(C, H, W, TH):
    HW = H * W
    hpW = (H + 2) * W
    THW = TH * W

    def _body(x_ref, w1_ref, b1_ref, w2_ref, b2_ref, wf_ref, bf_ref, be_ref,
              eta_ref, up_ref, h2_ref, acc_ref):
        t = pl.program_id(1)

        def convT(xp, w_ref, b_ref, m0, m2):
            zcol = jnp.zeros((C, 1), jnp.bfloat16)
            xr = jnp.concatenate([zcol, xp[:, :hpW - 1]], axis=1)
            xr = jnp.where(m0, xr, jnp.bfloat16(0))
            xl = jnp.concatenate([xp[:, 1:], zcol], axis=1)
            xl = jnp.where(m2, xl, jnp.bfloat16(0))
            xs = jnp.concatenate([xr, xp, xl], axis=0)
            acc = jnp.zeros((C, HW), jnp.float32)
            for dy in range(3):
                acc = acc + jnp.dot(w_ref[dy],
                                    xs[:, dy * W:dy * W + HW],
                                    preferred_element_type=jnp.float32)
            return acc + b_ref[...]

        @pl.when(t <= 1)
        def _():
            posw = lax.broadcasted_iota(jnp.int32, (1, hpW), 1) % W
            m0 = posw != 0
            m2 = posw != W - 1
            zrow = jnp.zeros((C, W), jnp.bfloat16)
            xf = x_ref[0, 0].astype(jnp.bfloat16).reshape(C, HW)
            xp = jnp.concatenate([zrow, xf, zrow], axis=1)
            h1 = jnp.maximum(convT(xp, w1_ref, b1_ref, m0, m2), 0.0)

            @pl.when(t == 0)
            def _():
                acc_ref[...] = h1

            @pl.when(t == 1)
            def _():
                gm = ((acc_ref[...] + h1) * 0.5).astype(jnp.bfloat16)
                gp = jnp.concatenate([zrow, gm, zrow], axis=1)
                h2 = jnp.maximum(convT(gp, w2_ref, b2_ref, m0, m2), 0.0)
                h2_ref[:, 0:W] = zrow
                h2_ref[:, W:W + HW] = h2.astype(jnp.bfloat16)
                h2_ref[:, W + HW:] = zrow

        @pl.when(t >= 1)
        def _():
            tt = t - 1
            r0W = pl.multiple_of(tt * THW, THW)
            hs = h2_ref[:, pl.ds(r0W, THW + 2 * W)]
            f = jnp.dot(wf_ref[...], hs,
                        preferred_element_type=jnp.float32) + bf_ref[...]
            up_ref[...] = f[:UPC, W:W + THW].reshape(1, 1, UPC, TH, W)

            E = f[UPC:UPC + 9, :]
            zc9 = jnp.zeros((9, 1), jnp.float32)
            Ep = jnp.concatenate([zc9, E, zc9], axis=1)
            lane = lax.broadcasted_iota(jnp.int32, (1, THW), 1) % W
            acc9 = jnp.zeros((1, THW), jnp.float32)
            for k in range(9):
                dy, dx = divmod(k, 3)
                s = Ep[k:k + 1, dy * W + dx:dy * W + dx + THW]
                if dx == 0:
                    s = jnp.where(lane != 0, s, 0.0)
                elif dx == 2:
                    s = jnp.where(lane != W - 1, s, 0.0)
                acc9 = acc9 + s
            z = acc9 + be_ref[0, 0]
            sp = jnp.where(z > 20.0, z,
                           jnp.log1p(jnp.exp(jnp.minimum(z, 20.0))))
            eta_ref[...] = (0.01 * sp).reshape(1, 1, TH, W)

    return _body


def kernel(net, w1, b1, w2, b2, we, be, wu, bu):
    B, num, C, H, W = net.shape
    G = num // 2
    BG = B * G
    TH = 32 if H % 32 == 0 else H
    NT = H // TH

    w1t = w1.reshape(3, 3, C, C).transpose(0, 3, 1, 2) \
            .reshape(3, C, 3 * C).astype(jnp.bfloat16)
    w2t = w2.reshape(3, 3, C, C).transpose(0, 3, 1, 2) \
            .reshape(3, C, 3 * C).astype(jnp.bfloat16)
    b1c = b1.reshape(C, 1)
    b2c = b2.reshape(C, 1)
    wft = jnp.zeros((CUF, C), jnp.float32)
    wft = wft.at[:UPC].set(wu.T).at[UPC:UPC + 9].set(we.reshape(9, C))
    wft = wft.astype(jnp.bfloat16)
    bfc = jnp.zeros((CUF, 1), jnp.float32).at[:UPC, 0].set(bu[0])

    eta, up = pl.pallas_call(
        _make_body(C, H, W, TH),
        out_shape=(
            jax.ShapeDtypeStruct((B, G, H, W), jnp.float32),
            jax.ShapeDtypeStruct((B, G, UPC, H, W), jnp.float32),
        ),
        grid=(BG, NT + 1),
        in_specs=[
            pl.BlockSpec(
                (1, 1, C, H, W),
                lambda g, t: (g // G, 2 * (g % G) + jnp.minimum(t, 1),
                              0, 0, 0)),
            pl.BlockSpec((3, C, 3 * C), lambda g, t: (0, 0, 0)),
            pl.BlockSpec((C, 1), lambda g, t: (0, 0)),
            pl.BlockSpec((3, C, 3 * C), lambda g, t: (0, 0, 0)),
            pl.BlockSpec((C, 1), lambda g, t: (0, 0)),
            pl.BlockSpec((CUF, C), lambda g, t: (0, 0)),
            pl.BlockSpec((CUF, 1), lambda g, t: (0, 0)),
            pl.BlockSpec((1, 1), lambda g, t: (0, 0)),
        ],
        out_specs=[
            pl.BlockSpec(
                (1, 1, TH, W),
                lambda g, t: (g // G, g % G, jnp.maximum(t - 1, 0), 0)),
            pl.BlockSpec(
                (1, 1, UPC, TH, W),
                lambda g, t: (g // G, g % G, 0, jnp.maximum(t - 1, 0), 0)),
        ],
        scratch_shapes=[pltpu.VMEM((C, (H + 2) * W), jnp.bfloat16),
                        pltpu.VMEM((C, H * W), jnp.float32)],
        compiler_params=pltpu.CompilerParams(
            dimension_semantics=("parallel", "arbitrary"),
            vmem_limit_bytes=VMEM_LIMIT),
    )(net, w1t, b1c, w2t, b2c, wft, bfc, be)

    return eta, up
